# baseline (speedup 1.0000x reference)
import math

import jax
import jax.numpy as jnp
from jax import lax
from jax.experimental import pallas as pl
from jax.experimental.pallas import tpu as pltpu

S = 4096
D = 2048
E = 64
CAP_F = 1.0
_capacity_fp = max(min(S, S / E * CAP_F), 4)
C = math.ceil(_capacity_fp)

T = 256
GRID = S // T


def _gate_body(x_ref, wg_ref, stats_ref, combine_ref, dmask_ref, carry_ref):
    i = pl.program_id(0)

    @pl.when(i == 0)
    def _init():
        carry_ref[...] = jnp.zeros_like(carry_ref)
        stats_ref[...] = jnp.zeros_like(stats_ref)

    x = x_ref[...]
    wg = wg_ref[...]
    logits = jnp.dot(x, wg, preferred_element_type=jnp.float32)

    lmax = jnp.max(logits, axis=1, keepdims=True)
    ex = jnp.exp(logits - lmax)
    sumex = jnp.sum(ex, axis=1, keepdims=True)
    gates = ex / sumex
    lse = lmax + jnp.log(sumex)

    gmax = jnp.max(gates, axis=1, keepdims=True)
    eidx = lax.broadcasted_iota(jnp.int32, (T, E), 1)
    e_s = jnp.min(jnp.where(gates == gmax, eidx, E), axis=1, keepdims=True)
    mask1 = (eidx == e_s).astype(jnp.float32)

    sum_gates_e = jnp.sum(gates, axis=0, keepdims=True)
    cnt_e = jnp.sum(mask1, axis=0, keepdims=True)
    sg = jnp.sum(gates, axis=1, keepdims=True)
    l2 = jnp.sqrt(jnp.sum(gates * gates, axis=1, keepdims=True))
    sl1_part = jnp.sum(sg / (l2 + 1e-9))
    ent_part = jnp.sum(-gates * jnp.log(gates + 1e-9))
    gs_part = jnp.sum(gmax)
    lse2_part = jnp.sum(lse * lse)

    r = lax.broadcasted_iota(jnp.int32, (T, T), 0)
    c = lax.broadcasted_iota(jnp.int32, (T, T), 1)
    tri = (r >= c).astype(jnp.float32)
    csum = jnp.dot(tri, mask1, preferred_element_type=jnp.float32)
    locations = carry_ref[...] + csum - 1.0
    carry_ref[...] = carry_ref[...] + cnt_e

    keep = (locations < float(C)).astype(jnp.float32)
    mask1k = mask1 * keep
    routed_e = jnp.sum(mask1k, axis=0, keepdims=True)

    loc_s = jnp.sum(locations * mask1k, axis=1, keepdims=True)
    gates1_s = jnp.sum(gates * mask1k, axis=1, keepdims=True)

    cidx = lax.broadcasted_iota(jnp.int32, (T, C), 1).astype(jnp.float32)
    onehot_c = (cidx == loc_s).astype(jnp.float32)
    gates1 = gates1_s * mask1k
    combine = gates1[:, :, None] * onehot_c[:, None, :]
    combine_ref[...] = combine
    dmask_ref[...] = combine != 0.0

    z64 = jnp.zeros((1, 64), jnp.float32)
    sidx = lax.broadcasted_iota(jnp.int32, (1, 128), 1)
    srow = (jnp.where(sidx == 0, sl1_part, 0.0)
            + jnp.where(sidx == 1, ent_part, 0.0)
            + jnp.where(sidx == 2, gs_part, 0.0)
            + jnp.where(sidx == 3, lse2_part, 0.0))
    contrib = jnp.concatenate([
        jnp.concatenate([sum_gates_e, z64], axis=1),
        jnp.concatenate([cnt_e, z64], axis=1),
        jnp.concatenate([routed_e, z64], axis=1),
        srow,
        jnp.zeros((4, 128), jnp.float32),
    ], axis=0)
    stats_ref[...] = stats_ref[...] + contrib


def kernel(x, wg):
    stats, combine, dmask = pl.pallas_call(
        _gate_body,
        grid=(GRID,),
        in_specs=[
            pl.BlockSpec((T, D), lambda i: (i, 0)),
            pl.BlockSpec((D, E), lambda i: (0, 0)),
        ],
        out_specs=[
            pl.BlockSpec((8, 128), lambda i: (0, 0)),
            pl.BlockSpec((T, E, C), lambda i: (i, 0, 0)),
            pl.BlockSpec((T, E, C), lambda i: (i, 0, 0)),
        ],
        out_shape=[
            jax.ShapeDtypeStruct((8, 128), jnp.float32),
            jax.ShapeDtypeStruct((S, E, C), jnp.float32),
            jax.ShapeDtypeStruct((S, E, C), jnp.bool_),
        ],
        scratch_shapes=[pltpu.VMEM((1, E), jnp.float32)],
    )(x, wg)

    sum_gates = stats[0, :E]
    cnt = stats[1, :E]
    routed = stats[2, :E]
    sf = jnp.float32(S)
    me = sum_gates / sf
    ce = cnt / sf
    l_aux0 = jnp.sum(me * ce) * E
    l_sl1 = stats[3, 0] / sf
    l_mil = jnp.sum(me * me) * E
    l_z = stats[3, 3] / sf
    batch_entropy = stats[3, 1] / sf
    batch_prob = stats[3, 2] / sf
    total_routed = jnp.sum(routed)
    fraction_routed = total_routed / sf
    expert_fraction = cnt / sf
    expert_fraction_routed = routed / total_routed
    l_aux = jnp.concatenate([
        jnp.stack([l_aux0, l_sl1, l_mil, l_z, batch_entropy, batch_prob,
                   fraction_routed]),
        expert_fraction,
        expert_fraction_routed,
    ])
    return (l_aux, combine, dmask, jnp.float32(_capacity_fp))


# speedup vs baseline: 3.8666x; 3.8666x over previous
import math

import jax
import jax.numpy as jnp
from jax import lax
from jax.experimental import pallas as pl
from jax.experimental.pallas import tpu as pltpu

S = 4096
D = 2048
E = 64
CAP_F = 1.0
_capacity_fp = max(min(S, S / E * CAP_F), 4)
C = math.ceil(_capacity_fp)

T = 256
GRID = S // T


def _gate_body(x_ref, wgt_ref, stats_ref, combine_ref, dmask_ref, carry_ref):
    i = pl.program_id(0)

    @pl.when(i == 0)
    def _init():
        carry_ref[...] = jnp.zeros_like(carry_ref)
        stats_ref[...] = jnp.zeros_like(stats_ref)

    x = x_ref[...]
    wgt = wgt_ref[...]
    logits = lax.dot_general(wgt, x, (((1,), (1,)), ((), ())),
                             preferred_element_type=jnp.float32)

    lmax = jnp.max(logits, axis=0, keepdims=True)
    ex = jnp.exp(logits - lmax)
    sumex = jnp.sum(ex, axis=0, keepdims=True)
    gates = ex / sumex
    lse = lmax + jnp.log(sumex)

    gmax = jnp.max(gates, axis=0, keepdims=True)
    eidx = lax.broadcasted_iota(jnp.int32, (E, T), 0)
    e_s = jnp.min(jnp.where(gates == gmax, eidx, E), axis=0, keepdims=True)
    mask1 = (eidx == e_s).astype(jnp.float32)

    sum_gates_e = jnp.sum(gates, axis=1, keepdims=True)
    cnt_e = jnp.sum(mask1, axis=1, keepdims=True)
    sg = jnp.sum(gates, axis=0, keepdims=True)
    l2 = jnp.sqrt(jnp.sum(gates * gates, axis=0, keepdims=True))
    sl1_part = jnp.sum(sg / (l2 + 1e-9))
    ent_part = jnp.sum(-gates * jnp.log(gates + 1e-9))
    gs_part = jnp.sum(gmax)
    lse2_part = jnp.sum(lse * lse)

    r = lax.broadcasted_iota(jnp.int32, (T, T), 0)
    c = lax.broadcasted_iota(jnp.int32, (T, T), 1)
    tri = (r <= c).astype(jnp.float32)
    csum = jnp.dot(mask1, tri, preferred_element_type=jnp.float32)
    locations = carry_ref[...] + csum - 1.0
    carry_ref[...] = carry_ref[...] + cnt_e

    keep = (locations < float(C)).astype(jnp.float32)
    mask1k = mask1 * keep
    routed_e = jnp.sum(mask1k, axis=1, keepdims=True)

    loc_s = jnp.sum(locations * mask1k, axis=0, keepdims=True)
    gates1_s = jnp.sum(gates * mask1k, axis=0, keepdims=True)

    cidx = lax.broadcasted_iota(jnp.int32, (C, T), 0).astype(jnp.float32)
    onehot_c = (cidx == loc_s).astype(jnp.float32)
    gates1 = gates1_s * mask1k
    combine = gates1[:, None, :] * onehot_c[None, :, :]
    combine_ref[...] = combine
    dmask_ref[...] = (combine != 0.0).astype(jnp.int8)

    ridx = lax.broadcasted_iota(jnp.int32, (E, 1), 0)
    svec = (jnp.where(ridx == 0, sl1_part, 0.0)
            + jnp.where(ridx == 1, ent_part, 0.0)
            + jnp.where(ridx == 2, gs_part, 0.0)
            + jnp.where(ridx == 3, lse2_part, 0.0))
    contrib = jnp.concatenate(
        [sum_gates_e, cnt_e, routed_e, svec,
         jnp.zeros((E, 4), jnp.float32)], axis=1)
    stats_ref[...] = stats_ref[...] + contrib


def kernel(x, wg):
    wgt = wg.T
    stats, combine_t, dmask_t = pl.pallas_call(
        _gate_body,
        grid=(GRID,),
        in_specs=[
            pl.BlockSpec((T, D), lambda i: (i, 0)),
            pl.BlockSpec((E, D), lambda i: (0, 0)),
        ],
        out_specs=[
            pl.BlockSpec((E, 8), lambda i: (0, 0)),
            pl.BlockSpec((E, C, T), lambda i: (0, 0, i)),
            pl.BlockSpec((E, C, T), lambda i: (0, 0, i)),
        ],
        out_shape=[
            jax.ShapeDtypeStruct((E, 8), jnp.float32),
            jax.ShapeDtypeStruct((E, C, S), jnp.float32),
            jax.ShapeDtypeStruct((E, C, S), jnp.int8),
        ],
        scratch_shapes=[pltpu.VMEM((E, 1), jnp.float32)],
    )(x, wgt)

    combine = jnp.transpose(combine_t, (2, 0, 1))
    dmask = jnp.transpose(dmask_t, (2, 0, 1)).astype(bool)

    sum_gates = stats[:, 0]
    cnt = stats[:, 1]
    routed = stats[:, 2]
    sf = jnp.float32(S)
    me = sum_gates / sf
    ce = cnt / sf
    l_aux0 = jnp.sum(me * ce) * E
    l_sl1 = stats[0, 3] / sf
    l_mil = jnp.sum(me * me) * E
    l_z = stats[3, 3] / sf
    batch_entropy = stats[1, 3] / sf
    batch_prob = stats[2, 3] / sf
    total_routed = jnp.sum(routed)
    fraction_routed = total_routed / sf
    expert_fraction = cnt / sf
    expert_fraction_routed = routed / total_routed
    l_aux = jnp.concatenate([
        jnp.stack([l_aux0, l_sl1, l_mil, l_z, batch_entropy, batch_prob,
                   fraction_routed]),
        expert_fraction,
        expert_fraction_routed,
    ])
    return (l_aux, combine, dmask, jnp.float32(_capacity_fp))


# speedup vs baseline: 4.1581x; 1.0754x over previous
import math

import jax
import jax.numpy as jnp
from jax import lax
from jax.experimental import pallas as pl
from jax.experimental.pallas import tpu as pltpu

S = 4096
D = 2048
E = 64
CAP_F = 1.0
_capacity_fp = max(min(S, S / E * CAP_F), 4)
C = math.ceil(_capacity_fp)

T = 512
GRID = S // T


def _gate_body(x_ref, wgt_ref, stats_ref, combine_ref, dmask_ref, carry_ref):
    i = pl.program_id(0)

    @pl.when(i == 0)
    def _init():
        carry_ref[...] = jnp.zeros_like(carry_ref)
        stats_ref[...] = jnp.zeros_like(stats_ref)

    x = x_ref[...]
    wgt = wgt_ref[...]
    logits = lax.dot_general(wgt, x, (((1,), (1,)), ((), ())),
                             preferred_element_type=jnp.float32)

    lmax = jnp.max(logits, axis=0, keepdims=True)
    ex = jnp.exp(logits - lmax)
    sumex = jnp.sum(ex, axis=0, keepdims=True)
    gates = ex / sumex
    lse = lmax + jnp.log(sumex)

    gmax = jnp.max(gates, axis=0, keepdims=True)
    eidx = lax.broadcasted_iota(jnp.int32, (E, T), 0)
    e_s = jnp.min(jnp.where(gates == gmax, eidx, E), axis=0, keepdims=True)
    mask1 = (eidx == e_s).astype(jnp.float32)

    sum_gates_e = jnp.sum(gates, axis=1, keepdims=True)
    cnt_e = jnp.sum(mask1, axis=1, keepdims=True)
    sg = jnp.sum(gates, axis=0, keepdims=True)
    l2 = jnp.sqrt(jnp.sum(gates * gates, axis=0, keepdims=True))
    sl1_part = jnp.sum(sg / (l2 + 1e-9))
    ent_part = jnp.sum(-gates * jnp.log(gates + 1e-9))
    gs_part = jnp.sum(gmax)
    lse2_part = jnp.sum(lse * lse)

    r = lax.broadcasted_iota(jnp.int32, (T, T), 0)
    c = lax.broadcasted_iota(jnp.int32, (T, T), 1)
    tri = (r <= c).astype(jnp.float32)
    csum = jnp.dot(mask1, tri, preferred_element_type=jnp.float32)
    locations = carry_ref[...] + csum - 1.0
    carry_ref[...] = carry_ref[...] + cnt_e

    keep = (locations < float(C)).astype(jnp.float32)
    mask1k = mask1 * keep
    routed_e = jnp.sum(mask1k, axis=1, keepdims=True)

    loc_s = jnp.sum(locations * mask1k, axis=0, keepdims=True)
    gates1_s = jnp.sum(gates * mask1k, axis=0, keepdims=True)

    cidx = lax.broadcasted_iota(jnp.int32, (C, T), 0).astype(jnp.float32)
    onehot_c = (cidx == loc_s).astype(jnp.float32)
    gates1 = gates1_s * mask1k
    combine = gates1[:, None, :] * onehot_c[None, :, :]
    combine_ref[...] = combine
    dmask_ref[...] = (combine != 0.0).astype(jnp.int8)

    ridx = lax.broadcasted_iota(jnp.int32, (E, 1), 0)
    svec = (jnp.where(ridx == 0, sl1_part, 0.0)
            + jnp.where(ridx == 1, ent_part, 0.0)
            + jnp.where(ridx == 2, gs_part, 0.0)
            + jnp.where(ridx == 3, lse2_part, 0.0))
    contrib = jnp.concatenate(
        [sum_gates_e, cnt_e, routed_e, svec,
         jnp.zeros((E, 4), jnp.float32)], axis=1)
    stats_ref[...] = stats_ref[...] + contrib


def kernel(x, wg):
    wgt = wg.T
    stats, combine_t, dmask_t = pl.pallas_call(
        _gate_body,
        grid=(GRID,),
        in_specs=[
            pl.BlockSpec((T, D), lambda i: (i, 0)),
            pl.BlockSpec((E, D), lambda i: (0, 0)),
        ],
        out_specs=[
            pl.BlockSpec((E, 8), lambda i: (0, 0)),
            pl.BlockSpec((E, C, T), lambda i: (0, 0, i)),
            pl.BlockSpec((E, C, T), lambda i: (0, 0, i)),
        ],
        out_shape=[
            jax.ShapeDtypeStruct((E, 8), jnp.float32),
            jax.ShapeDtypeStruct((E, C, S), jnp.float32),
            jax.ShapeDtypeStruct((E, C, S), jnp.int8),
        ],
        scratch_shapes=[pltpu.VMEM((E, 1), jnp.float32)],
    )(x, wgt)

    combine = jnp.transpose(combine_t, (2, 0, 1))
    dmask = jnp.transpose(dmask_t, (2, 0, 1)).astype(bool)

    sum_gates = stats[:, 0]
    cnt = stats[:, 1]
    routed = stats[:, 2]
    sf = jnp.float32(S)
    me = sum_gates / sf
    ce = cnt / sf
    l_aux0 = jnp.sum(me * ce) * E
    l_sl1 = stats[0, 3] / sf
    l_mil = jnp.sum(me * me) * E
    l_z = stats[3, 3] / sf
    batch_entropy = stats[1, 3] / sf
    batch_prob = stats[2, 3] / sf
    total_routed = jnp.sum(routed)
    fraction_routed = total_routed / sf
    expert_fraction = cnt / sf
    expert_fraction_routed = routed / total_routed
    l_aux = jnp.concatenate([
        jnp.stack([l_aux0, l_sl1, l_mil, l_z, batch_entropy, batch_prob,
                   fraction_routed]),
        expert_fraction,
        expert_fraction_routed,
    ])
    return (l_aux, combine, dmask, jnp.float32(_capacity_fp))


# speedup vs baseline: 4.6875x; 1.1273x over previous
import math

import jax
import jax.numpy as jnp
from jax import lax
from jax.experimental import pallas as pl
from jax.experimental.pallas import tpu as pltpu

S = 4096
D = 2048
E = 64
CAP_F = 1.0
_capacity_fp = max(min(S, S / E * CAP_F), 4)
C = math.ceil(_capacity_fp)

T = 512
GRID = S // T


def _gate_body(x_ref, wgt_ref, stats_ref, combine_ref, fidx_ref, carry_ref):
    i = pl.program_id(0)

    @pl.when(i == 0)
    def _init():
        carry_ref[...] = jnp.zeros_like(carry_ref)
        stats_ref[...] = jnp.zeros_like(stats_ref)

    x = x_ref[...]
    wgt = wgt_ref[...]
    logits = lax.dot_general(wgt, x, (((1,), (1,)), ((), ())),
                             preferred_element_type=jnp.float32)

    lmax = jnp.max(logits, axis=0, keepdims=True)
    ex = jnp.exp(logits - lmax)
    sumex = jnp.sum(ex, axis=0, keepdims=True)
    gates = ex / sumex
    lse = lmax + jnp.log(sumex)

    gmax = jnp.max(gates, axis=0, keepdims=True)
    eidx = lax.broadcasted_iota(jnp.int32, (E, T), 0)
    e_s = jnp.min(jnp.where(gates == gmax, eidx, E), axis=0, keepdims=True)
    mask1 = (eidx == e_s).astype(jnp.float32)

    sum_gates_e = jnp.sum(gates, axis=1, keepdims=True)
    cnt_e = jnp.sum(mask1, axis=1, keepdims=True)
    sg = jnp.sum(gates, axis=0, keepdims=True)
    l2 = jnp.sqrt(jnp.sum(gates * gates, axis=0, keepdims=True))
    sl1_part = jnp.sum(sg / (l2 + 1e-9))
    ent_part = jnp.sum(-gates * jnp.log(gates + 1e-9))
    gs_part = jnp.sum(gmax)
    lse2_part = jnp.sum(lse * lse)

    r = lax.broadcasted_iota(jnp.int32, (T, T), 0)
    c = lax.broadcasted_iota(jnp.int32, (T, T), 1)
    tri = (r <= c).astype(jnp.float32)
    csum = jnp.dot(mask1, tri, preferred_element_type=jnp.float32)
    locations = carry_ref[...] + csum - 1.0
    carry_ref[...] = carry_ref[...] + cnt_e

    keep = (locations < float(C)).astype(jnp.float32)
    mask1k = mask1 * keep
    routed_e = jnp.sum(mask1k, axis=1, keepdims=True)

    loc_s = jnp.sum(locations * mask1k, axis=0, keepdims=True)
    gates1_s = jnp.sum(gates * mask1k, axis=0, keepdims=True)

    cidx = lax.broadcasted_iota(jnp.int32, (C, T), 0).astype(jnp.float32)
    onehot_c = (cidx == loc_s).astype(jnp.float32)
    gates1 = gates1_s * mask1k
    combine = gates1[:, None, :] * onehot_c[None, :, :]
    combine_ref[...] = combine

    ksum = jnp.sum(mask1k, axis=0, keepdims=True)
    flat = e_s * C + loc_s.astype(jnp.int32)
    fidx_ref[...] = jnp.where(ksum > 0.0, flat, -1)

    ridx = lax.broadcasted_iota(jnp.int32, (E, 1), 0)
    svec = (jnp.where(ridx == 0, sl1_part, 0.0)
            + jnp.where(ridx == 1, ent_part, 0.0)
            + jnp.where(ridx == 2, gs_part, 0.0)
            + jnp.where(ridx == 3, lse2_part, 0.0))
    contrib = jnp.concatenate(
        [sum_gates_e, cnt_e, routed_e, svec,
         jnp.zeros((E, 4), jnp.float32)], axis=1)
    stats_ref[...] = stats_ref[...] + contrib


def kernel(x, wg):
    wgt = wg.T
    stats, combine_t, fidx = pl.pallas_call(
        _gate_body,
        grid=(GRID,),
        in_specs=[
            pl.BlockSpec((T, D), lambda i: (i, 0)),
            pl.BlockSpec((E, D), lambda i: (0, 0)),
        ],
        out_specs=[
            pl.BlockSpec((E, 8), lambda i: (0, 0)),
            pl.BlockSpec((E, C, T), lambda i: (0, 0, i)),
            pl.BlockSpec((1, T), lambda i: (0, i)),
        ],
        out_shape=[
            jax.ShapeDtypeStruct((E, 8), jnp.float32),
            jax.ShapeDtypeStruct((E, C, S), jnp.float32),
            jax.ShapeDtypeStruct((1, S), jnp.int32),
        ],
        scratch_shapes=[pltpu.VMEM((E, 1), jnp.float32)],
    )(x, wgt)

    combine = jnp.transpose(combine_t, (2, 0, 1))
    ec = (jnp.arange(E, dtype=jnp.int32)[:, None] * C
          + jnp.arange(C, dtype=jnp.int32)[None, :])
    dmask = fidx[0][:, None, None] == ec[None, :, :]

    sum_gates = stats[:, 0]
    cnt = stats[:, 1]
    routed = stats[:, 2]
    sf = jnp.float32(S)
    me = sum_gates / sf
    ce = cnt / sf
    l_aux0 = jnp.sum(me * ce) * E
    l_sl1 = stats[0, 3] / sf
    l_mil = jnp.sum(me * me) * E
    l_z = stats[3, 3] / sf
    batch_entropy = stats[1, 3] / sf
    batch_prob = stats[2, 3] / sf
    total_routed = jnp.sum(routed)
    fraction_routed = total_routed / sf
    expert_fraction = cnt / sf
    expert_fraction_routed = routed / total_routed
    l_aux = jnp.concatenate([
        jnp.stack([l_aux0, l_sl1, l_mil, l_z, batch_entropy, batch_prob,
                   fraction_routed]),
        expert_fraction,
        expert_fraction_routed,
    ])
    return (l_aux, combine, dmask, jnp.float32(_capacity_fp))


# speedup vs baseline: 4.8192x; 1.0281x over previous
import math

import jax
import jax.numpy as jnp
from jax import lax
from jax.experimental import pallas as pl
from jax.experimental.pallas import tpu as pltpu

S = 4096
D = 2048
E = 64
CAP_F = 1.0
_capacity_fp = max(min(S, S / E * CAP_F), 4)
C = math.ceil(_capacity_fp)

T = 1024
GRID = S // T


def _gate_body(x_ref, wgt_ref, stats_ref, combine_ref, fidx_ref, carry_ref):
    i = pl.program_id(0)

    @pl.when(i == 0)
    def _init():
        carry_ref[...] = jnp.zeros_like(carry_ref)
        stats_ref[...] = jnp.zeros_like(stats_ref)

    x = x_ref[...]
    wgt = wgt_ref[...]
    logits = lax.dot_general(wgt, x, (((1,), (1,)), ((), ())),
                             preferred_element_type=jnp.float32)

    lmax = jnp.max(logits, axis=0, keepdims=True)
    ex = jnp.exp(logits - lmax)
    sumex = jnp.sum(ex, axis=0, keepdims=True)
    gates = ex / sumex
    lse = lmax + jnp.log(sumex)

    gmax = jnp.max(gates, axis=0, keepdims=True)
    eidx = lax.broadcasted_iota(jnp.int32, (E, T), 0)
    e_s = jnp.min(jnp.where(gates == gmax, eidx, E), axis=0, keepdims=True)
    mask1 = (eidx == e_s).astype(jnp.float32)

    sum_gates_e = jnp.sum(gates, axis=1, keepdims=True)
    cnt_e = jnp.sum(mask1, axis=1, keepdims=True)
    sg = jnp.sum(gates, axis=0, keepdims=True)
    l2 = jnp.sqrt(jnp.sum(gates * gates, axis=0, keepdims=True))
    sl1_part = jnp.sum(sg / (l2 + 1e-9))
    ent_part = jnp.sum(-gates * jnp.log(gates + 1e-9))
    gs_part = jnp.sum(gmax)
    lse2_part = jnp.sum(lse * lse)

    r = lax.broadcasted_iota(jnp.int32, (T, T), 0)
    c = lax.broadcasted_iota(jnp.int32, (T, T), 1)
    tri = (r <= c).astype(jnp.float32)
    csum = jnp.dot(mask1, tri, preferred_element_type=jnp.float32)
    locations = carry_ref[...] + csum - 1.0
    carry_ref[...] = carry_ref[...] + cnt_e

    keep = (locations < float(C)).astype(jnp.float32)
    mask1k = mask1 * keep
    routed_e = jnp.sum(mask1k, axis=1, keepdims=True)

    loc_s = jnp.sum(locations * mask1k, axis=0, keepdims=True)
    gates1_s = jnp.sum(gates * mask1k, axis=0, keepdims=True)

    cidx = lax.broadcasted_iota(jnp.int32, (C, T), 0).astype(jnp.float32)
    onehot_c = (cidx == loc_s).astype(jnp.float32)
    gates1 = gates1_s * mask1k
    combine = gates1[:, None, :] * onehot_c[None, :, :]
    combine_ref[...] = combine

    ksum = jnp.sum(mask1k, axis=0, keepdims=True)
    flat = e_s * C + loc_s.astype(jnp.int32)
    fidx_ref[...] = jnp.where(ksum > 0.0, flat, -1)

    ridx = lax.broadcasted_iota(jnp.int32, (E, 1), 0)
    svec = (jnp.where(ridx == 0, sl1_part, 0.0)
            + jnp.where(ridx == 1, ent_part, 0.0)
            + jnp.where(ridx == 2, gs_part, 0.0)
            + jnp.where(ridx == 3, lse2_part, 0.0))
    contrib = jnp.concatenate(
        [sum_gates_e, cnt_e, routed_e, svec,
         jnp.zeros((E, 4), jnp.float32)], axis=1)
    stats_ref[...] = stats_ref[...] + contrib


def kernel(x, wg):
    wgt = wg.T
    stats, combine_t, fidx = pl.pallas_call(
        _gate_body,
        grid=(GRID,),
        in_specs=[
            pl.BlockSpec((T, D), lambda i: (i, 0)),
            pl.BlockSpec((E, D), lambda i: (0, 0)),
        ],
        out_specs=[
            pl.BlockSpec((E, 8), lambda i: (0, 0)),
            pl.BlockSpec((E, C, T), lambda i: (0, 0, i)),
            pl.BlockSpec((1, T), lambda i: (0, i)),
        ],
        out_shape=[
            jax.ShapeDtypeStruct((E, 8), jnp.float32),
            jax.ShapeDtypeStruct((E, C, S), jnp.float32),
            jax.ShapeDtypeStruct((1, S), jnp.int32),
        ],
        scratch_shapes=[pltpu.VMEM((E, 1), jnp.float32)],
    )(x, wgt)

    combine = jnp.transpose(combine_t, (2, 0, 1))
    ec = (jnp.arange(E, dtype=jnp.int32)[:, None] * C
          + jnp.arange(C, dtype=jnp.int32)[None, :])
    dmask = fidx[0][:, None, None] == ec[None, :, :]

    sum_gates = stats[:, 0]
    cnt = stats[:, 1]
    routed = stats[:, 2]
    sf = jnp.float32(S)
    me = sum_gates / sf
    ce = cnt / sf
    l_aux0 = jnp.sum(me * ce) * E
    l_sl1 = stats[0, 3] / sf
    l_mil = jnp.sum(me * me) * E
    l_z = stats[3, 3] / sf
    batch_entropy = stats[1, 3] / sf
    batch_prob = stats[2, 3] / sf
    total_routed = jnp.sum(routed)
    fraction_routed = total_routed / sf
    expert_fraction = cnt / sf
    expert_fraction_routed = routed / total_routed
    l_aux = jnp.concatenate([
        jnp.stack([l_aux0, l_sl1, l_mil, l_z, batch_entropy, batch_prob,
                   fraction_routed]),
        expert_fraction,
        expert_fraction_routed,
    ])
    return (l_aux, combine, dmask, jnp.float32(_capacity_fp))
